# baseline (speedup 1.0000x reference)
import jax
import jax.numpy as jnp
from jax.experimental import pallas as pl
from jax.experimental.pallas import tpu as pltpu

_N = 16384
_C = 4096
_K = 8
_D = 64
_NB = 256
_LANES = 128
_NCH = _C // _LANES


def _sort4_bitonic(v):
    a0 = jnp.minimum(v[0], v[2])
    a1 = jnp.minimum(v[1], v[3])
    a2 = jnp.maximum(v[0], v[2])
    a3 = jnp.maximum(v[1], v[3])
    return (jnp.minimum(a0, a1), jnp.maximum(a0, a1),
            jnp.minimum(a2, a3), jnp.maximum(a2, a3))


def _sort8_bitonic(v):
    lo = [jnp.minimum(v[i], v[i + 4]) for i in range(4)]
    hi = [jnp.maximum(v[i], v[i + 4]) for i in range(4)]
    return _sort4_bitonic(lo) + _sort4_bitonic(hi)


def _merge22(a0, a1, b0, b1):
    c0 = jnp.minimum(a0, b0)
    t1 = jnp.maximum(a0, b0)
    t2 = jnp.minimum(a1, b1)
    c3 = jnp.maximum(a1, b1)
    return (c0, jnp.minimum(t1, t2), jnp.maximum(t1, t2), c3)


def _merge44(a, b):
    e = _merge22(a[0], a[2], b[0], b[2])
    o = _merge22(a[1], a[3], b[1], b[3])
    c1 = jnp.minimum(e[1], o[0])
    c2 = jnp.maximum(e[1], o[0])
    c3 = jnp.minimum(e[2], o[1])
    c4 = jnp.maximum(e[2], o[1])
    c5 = jnp.minimum(e[3], o[2])
    c6 = jnp.maximum(e[3], o[2])
    return (e[0], c1, c2, c3, c4, c5, c6, o[3])


def _low8(a, b):
    return [jnp.minimum(a[i], b[7 - i]) for i in range(8)]


def _low4(a, b):
    return [jnp.minimum(a[i], b[3 - i]) for i in range(4)]


def _top8_threshold(d_sel):
    cols = [d_sel[:, i * _LANES:(i + 1) * _LANES] for i in range(_NCH)]
    s2 = []
    for i in range(16):
        a, b = cols[2 * i], cols[2 * i + 1]
        s2.append((jnp.minimum(a, b), jnp.maximum(a, b)))
    s4 = [_merge22(*s2[2 * i], *s2[2 * i + 1]) for i in range(8)]
    f4 = [_sort4_bitonic(_low4(s4[2 * i], s4[2 * i + 1])) for i in range(4)]
    f2 = [_sort4_bitonic(_low4(f4[0], f4[1])),
          _sort4_bitonic(_low4(f4[2], f4[3]))]
    cand = _low4(f2[0], f2[1])
    m = cand[0]
    for c in cand[1:]:
        m = jnp.minimum(m, c)
    t = jnp.min(m, axis=1, keepdims=True)
    for _ in range(_K - 1):
        masked = [jnp.where(c > t, c, jnp.inf) for c in cand]
        m = masked[0]
        for c in masked[1:]:
            m = jnp.minimum(m, c)
        t = jnp.min(m, axis=1, keepdims=True)
    return t


def _tc_body(idx_ref, q_ref, cptm2_ref, c2_ref, codes_ref, out_ref):
    q = q_ref[...]
    qx, qy, qz = q[:, 0:1], q[:, 1:2], q[:, 2:3]
    cm2 = cptm2_ref[0]

    s_qc = jnp.dot(q, cm2, preferred_element_type=jnp.float32)
    q2 = jnp.sum(q * q, axis=1, keepdims=True)
    a = q2 + c2_ref[0]
    d_sel = a + s_qc

    b2 = qx * cm2[0:1, :] + qy * cm2[1:2, :] + qz * cm2[2:3, :]
    d = a + b2

    t = _top8_threshold(d_sel)

    w = jnp.where(d_sel <= t, 1.0 / (d + 1e-16), 0.0)
    acc = jnp.dot(w, codes_ref[0], preferred_element_type=jnp.float32)
    out_ref[...] = acc[:, :_D] / acc[:, _D:_D + 1]


def _run(indices, q2d, cptm2, c2, codes):
    grid = (_N // _NB,)
    return pl.pallas_call(
        _tc_body,
        grid_spec=pltpu.PrefetchScalarGridSpec(
            num_scalar_prefetch=1,
            grid=grid,
            in_specs=[
                pl.BlockSpec((_NB, 3), lambda i, idx: (i, 0)),
                pl.BlockSpec((1, 3, _C), lambda i, idx: (idx[0], 0, 0)),
                pl.BlockSpec((1, 1, _C), lambda i, idx: (idx[0], 0, 0)),
                pl.BlockSpec((1, _C, _D + 1), lambda i, idx: (idx[0], 0, 0)),
            ],
            out_specs=pl.BlockSpec((_NB, _D), lambda i, idx: (i, 0)),
        ),
        out_shape=jax.ShapeDtypeStruct((_N, _D), jnp.float32),
    )(indices, q2d, cptm2, c2, codes)


def kernel(indices, query_points, codes_position, codes):
    q2d = query_points[0]
    cpt = jnp.transpose(codes_position, (0, 2, 1))
    cptm2 = -2.0 * cpt
    c2 = jnp.sum(cpt * cpt, axis=1)[:, None, :]
    ones = jnp.ones(codes.shape[:-1] + (1,), codes.dtype)
    codes_ext = jnp.concatenate([codes, ones], axis=-1)
    return _run(indices.astype(jnp.int32), q2d, cptm2, c2, codes_ext)


# speedup vs baseline: 1.0687x; 1.0687x over previous
import jax
import jax.numpy as jnp
from jax.experimental import pallas as pl
from jax.experimental.pallas import tpu as pltpu

_N = 16384
_C = 4096
_K = 8
_D = 64
_NB = 1024
_LANES = 128
_NCH = _C // _LANES


def _sort4_bitonic(v):
    a0 = jnp.minimum(v[0], v[2])
    a1 = jnp.minimum(v[1], v[3])
    a2 = jnp.maximum(v[0], v[2])
    a3 = jnp.maximum(v[1], v[3])
    return (jnp.minimum(a0, a1), jnp.maximum(a0, a1),
            jnp.minimum(a2, a3), jnp.maximum(a2, a3))


def _sort8_bitonic(v):
    lo = [jnp.minimum(v[i], v[i + 4]) for i in range(4)]
    hi = [jnp.maximum(v[i], v[i + 4]) for i in range(4)]
    return _sort4_bitonic(lo) + _sort4_bitonic(hi)


def _merge22(a0, a1, b0, b1):
    c0 = jnp.minimum(a0, b0)
    t1 = jnp.maximum(a0, b0)
    t2 = jnp.minimum(a1, b1)
    c3 = jnp.maximum(a1, b1)
    return (c0, jnp.minimum(t1, t2), jnp.maximum(t1, t2), c3)


def _merge44(a, b):
    e = _merge22(a[0], a[2], b[0], b[2])
    o = _merge22(a[1], a[3], b[1], b[3])
    c1 = jnp.minimum(e[1], o[0])
    c2 = jnp.maximum(e[1], o[0])
    c3 = jnp.minimum(e[2], o[1])
    c4 = jnp.maximum(e[2], o[1])
    c5 = jnp.minimum(e[3], o[2])
    c6 = jnp.maximum(e[3], o[2])
    return (e[0], c1, c2, c3, c4, c5, c6, o[3])


def _low8(a, b):
    return [jnp.minimum(a[i], b[7 - i]) for i in range(8)]


def _low4(a, b):
    return [jnp.minimum(a[i], b[3 - i]) for i in range(4)]


def _top8_threshold(d_sel):
    cols = [d_sel[:, i * _LANES:(i + 1) * _LANES] for i in range(_NCH)]
    s2 = []
    for i in range(16):
        a, b = cols[2 * i], cols[2 * i + 1]
        s2.append((jnp.minimum(a, b), jnp.maximum(a, b)))
    s4 = [_merge22(*s2[2 * i], *s2[2 * i + 1]) for i in range(8)]
    f4 = [_sort4_bitonic(_low4(s4[2 * i], s4[2 * i + 1])) for i in range(4)]
    f2 = [_sort4_bitonic(_low4(f4[0], f4[1])),
          _sort4_bitonic(_low4(f4[2], f4[3]))]
    cand = _low4(f2[0], f2[1])
    m = cand[0]
    for c in cand[1:]:
        m = jnp.minimum(m, c)
    t = jnp.min(m, axis=1, keepdims=True)
    for _ in range(_K - 1):
        masked = [jnp.where(c > t, c, jnp.inf) for c in cand]
        m = masked[0]
        for c in masked[1:]:
            m = jnp.minimum(m, c)
        t = jnp.min(m, axis=1, keepdims=True)
    return t


def _tc_body(idx_ref, q_ref, cptm2_ref, c2_ref, codes_ref, out_ref):
    q = q_ref[...]
    qx, qy, qz = q[:, 0:1], q[:, 1:2], q[:, 2:3]
    cm2 = cptm2_ref[0]

    s_qc = jnp.dot(q, cm2, preferred_element_type=jnp.float32)
    q2 = jnp.sum(q * q, axis=1, keepdims=True)
    a = q2 + c2_ref[0]
    d_sel = a + s_qc

    b2 = qx * cm2[0:1, :] + qy * cm2[1:2, :] + qz * cm2[2:3, :]
    d = a + b2

    t = _top8_threshold(d_sel)

    w = jnp.where(d_sel <= t, 1.0 / (d + 1e-16), 0.0)
    acc = jnp.dot(w, codes_ref[0], preferred_element_type=jnp.float32)
    out_ref[...] = acc[:, :_D] / acc[:, _D:_D + 1]


def _run(indices, q2d, cptm2, c2, codes):
    grid = (_N // _NB,)
    return pl.pallas_call(
        _tc_body,
        grid_spec=pltpu.PrefetchScalarGridSpec(
            num_scalar_prefetch=1,
            grid=grid,
            in_specs=[
                pl.BlockSpec((_NB, 3), lambda i, idx: (i, 0)),
                pl.BlockSpec((1, 3, _C), lambda i, idx: (idx[0], 0, 0)),
                pl.BlockSpec((1, 1, _C), lambda i, idx: (idx[0], 0, 0)),
                pl.BlockSpec((1, _C, _D + 1), lambda i, idx: (idx[0], 0, 0)),
            ],
            out_specs=pl.BlockSpec((_NB, _D), lambda i, idx: (i, 0)),
        ),
        out_shape=jax.ShapeDtypeStruct((_N, _D), jnp.float32),
    )(indices, q2d, cptm2, c2, codes)


def kernel(indices, query_points, codes_position, codes):
    q2d = query_points[0]
    cpt = jnp.transpose(codes_position, (0, 2, 1))
    cptm2 = -2.0 * cpt
    c2 = jnp.sum(cpt * cpt, axis=1)[:, None, :]
    ones = jnp.ones(codes.shape[:-1] + (1,), codes.dtype)
    codes_ext = jnp.concatenate([codes, ones], axis=-1)
    return _run(indices.astype(jnp.int32), q2d, cptm2, c2, codes_ext)


# speedup vs baseline: 1.0906x; 1.0205x over previous
import jax
import jax.numpy as jnp
from jax.experimental import pallas as pl
from jax.experimental.pallas import tpu as pltpu

_N = 16384
_C = 4096
_K = 8
_D = 64
_NB = 1024
_LANES = 128
_NCH = _C // _LANES


def _sort4_bitonic(v):
    a0 = jnp.minimum(v[0], v[2])
    a1 = jnp.minimum(v[1], v[3])
    a2 = jnp.maximum(v[0], v[2])
    a3 = jnp.maximum(v[1], v[3])
    return (jnp.minimum(a0, a1), jnp.maximum(a0, a1),
            jnp.minimum(a2, a3), jnp.maximum(a2, a3))


def _merge22(a0, a1, b0, b1):
    c0 = jnp.minimum(a0, b0)
    t1 = jnp.maximum(a0, b0)
    t2 = jnp.minimum(a1, b1)
    c3 = jnp.maximum(a1, b1)
    return (c0, jnp.minimum(t1, t2), jnp.maximum(t1, t2), c3)


def _low4(a, b):
    return [jnp.minimum(a[i], b[3 - i]) for i in range(4)]


def _top8_threshold(d_sel):
    cols = [d_sel[:, i * _LANES:(i + 1) * _LANES] for i in range(_NCH)]
    s2 = []
    for i in range(16):
        a, b = cols[2 * i], cols[2 * i + 1]
        s2.append((jnp.minimum(a, b), jnp.maximum(a, b)))
    s4 = [_merge22(*s2[2 * i], *s2[2 * i + 1]) for i in range(8)]
    f4 = [_sort4_bitonic(_low4(s4[2 * i], s4[2 * i + 1])) for i in range(4)]
    f2 = [_sort4_bitonic(_low4(f4[0], f4[1])),
          _sort4_bitonic(_low4(f4[2], f4[3]))]
    cand = _low4(f2[0], f2[1])
    m = cand[0]
    for c in cand[1:]:
        m = jnp.minimum(m, c)
    t = jnp.min(m, axis=1, keepdims=True)
    for _ in range(_K - 1):
        masked = [jnp.where(c > t, c, jnp.inf) for c in cand]
        m = masked[0]
        for c in masked[1:]:
            m = jnp.minimum(m, c)
        t = jnp.min(m, axis=1, keepdims=True)
    return t


def _prep_body(idx_ref, cp_ref, codes_ref, cptm2_ref, c2_ref, codes_ext_ref):
    cpt = cp_ref[0].T
    cptm2_ref[...] = -2.0 * cpt
    c2_ref[...] = jnp.sum(cpt * cpt, axis=0, keepdims=True)
    codes_ext_ref[:, :_D] = codes_ref[0]
    codes_ext_ref[:, _D:_D + 1] = jnp.ones((_C, 1), jnp.float32)


def _main_body(q_ref, cptm2_ref, c2_ref, codes_ref, out_ref):
    q = q_ref[...]
    qx, qy, qz = q[:, 0:1], q[:, 1:2], q[:, 2:3]
    cm2 = cptm2_ref[...]

    s_qc = jnp.dot(q, cm2, preferred_element_type=jnp.float32)
    q2 = jnp.sum(q * q, axis=1, keepdims=True)
    a = q2 + c2_ref[...]
    d_sel = a + s_qc

    b2 = qx * cm2[0:1, :] + qy * cm2[1:2, :] + qz * cm2[2:3, :]
    d = a + b2

    t = _top8_threshold(d_sel)

    w = jnp.where(d_sel <= t, 1.0 / (d + 1e-16), 0.0)
    acc = jnp.dot(w, codes_ref[...], preferred_element_type=jnp.float32)
    out_ref[...] = acc[:, :_D] / acc[:, _D:_D + 1]


def _prep(indices, codes_position, codes):
    return pl.pallas_call(
        _prep_body,
        grid_spec=pltpu.PrefetchScalarGridSpec(
            num_scalar_prefetch=1,
            grid=(1,),
            in_specs=[
                pl.BlockSpec((1, _C, 3), lambda i, idx: (idx[0], 0, 0)),
                pl.BlockSpec((1, _C, _D), lambda i, idx: (idx[0], 0, 0)),
            ],
            out_specs=[
                pl.BlockSpec((3, _C), lambda i, idx: (0, 0)),
                pl.BlockSpec((1, _C), lambda i, idx: (0, 0)),
                pl.BlockSpec((_C, _D + 1), lambda i, idx: (0, 0)),
            ],
        ),
        out_shape=[
            jax.ShapeDtypeStruct((3, _C), jnp.float32),
            jax.ShapeDtypeStruct((1, _C), jnp.float32),
            jax.ShapeDtypeStruct((_C, _D + 1), jnp.float32),
        ],
    )(indices, codes_position, codes)


def _main(q2d, cptm2, c2, codes_ext):
    return pl.pallas_call(
        _main_body,
        grid=(_N // _NB,),
        in_specs=[
            pl.BlockSpec((_NB, 3), lambda i: (i, 0)),
            pl.BlockSpec((3, _C), lambda i: (0, 0)),
            pl.BlockSpec((1, _C), lambda i: (0, 0)),
            pl.BlockSpec((_C, _D + 1), lambda i: (0, 0)),
        ],
        out_specs=pl.BlockSpec((_NB, _D), lambda i: (i, 0)),
        out_shape=jax.ShapeDtypeStruct((_N, _D), jnp.float32),
    )(q2d, cptm2, c2, codes_ext)


def kernel(indices, query_points, codes_position, codes):
    cptm2, c2, codes_ext = _prep(indices, codes_position, codes)
    return _main(query_points[0], cptm2, c2, codes_ext)


# speedup vs baseline: 1.1368x; 1.0424x over previous
import jax
import jax.numpy as jnp
from jax.experimental import pallas as pl
from jax.experimental.pallas import tpu as pltpu

_N = 16384
_C = 4096
_K = 8
_D = 64
_NB = 1024
_LANES = 128
_NCH = _C // _LANES


def _sort4_bitonic(v):
    a0 = jnp.minimum(v[0], v[2])
    a1 = jnp.minimum(v[1], v[3])
    a2 = jnp.maximum(v[0], v[2])
    a3 = jnp.maximum(v[1], v[3])
    return (jnp.minimum(a0, a1), jnp.maximum(a0, a1),
            jnp.minimum(a2, a3), jnp.maximum(a2, a3))


def _merge22(a0, a1, b0, b1):
    c0 = jnp.minimum(a0, b0)
    t1 = jnp.maximum(a0, b0)
    t2 = jnp.minimum(a1, b1)
    c3 = jnp.maximum(a1, b1)
    return (c0, jnp.minimum(t1, t2), jnp.maximum(t1, t2), c3)


def _low4(a, b):
    return [jnp.minimum(a[i], b[3 - i]) for i in range(4)]


def _top8_threshold(d_sel):
    cols = [d_sel[:, i * _LANES:(i + 1) * _LANES] for i in range(_NCH)]
    s2 = []
    for i in range(16):
        a, b = cols[2 * i], cols[2 * i + 1]
        s2.append((jnp.minimum(a, b), jnp.maximum(a, b)))
    s4 = [_merge22(*s2[2 * i], *s2[2 * i + 1]) for i in range(8)]
    f4 = [_sort4_bitonic(_low4(s4[2 * i], s4[2 * i + 1])) for i in range(4)]
    f2 = [_sort4_bitonic(_low4(f4[0], f4[1])),
          _sort4_bitonic(_low4(f4[2], f4[3]))]
    cand = _low4(f2[0], f2[1])
    m = cand[0]
    for c in cand[1:]:
        m = jnp.minimum(m, c)
    t = jnp.min(m, axis=1, keepdims=True)
    for _ in range(_K - 1):
        masked = [jnp.where(c > t, c, jnp.inf) for c in cand]
        m = masked[0]
        for c in masked[1:]:
            m = jnp.minimum(m, c)
        t = jnp.min(m, axis=1, keepdims=True)
    return t


def _body(idx_ref, q_ref, cp_ref, codes_ref, out_ref):
    cpt = cp_ref[0].T
    cm2 = -2.0 * cpt
    c2 = jnp.sum(cpt * cpt, axis=0, keepdims=True)

    q = q_ref[...]
    qx, qy, qz = q[:, 0:1], q[:, 1:2], q[:, 2:3]

    s_qc = jnp.dot(q, cm2, preferred_element_type=jnp.float32)
    q2 = jnp.sum(q * q, axis=1, keepdims=True)
    a = q2 + c2
    d_sel = a + s_qc

    b2 = qx * cm2[0:1, :] + qy * cm2[1:2, :] + qz * cm2[2:3, :]
    d = a + b2

    t = _top8_threshold(d_sel)

    w = jnp.where(d_sel <= t, 1.0 / (d + 1e-16), 0.0)
    acc = jnp.dot(w, codes_ref[0], preferred_element_type=jnp.float32)
    s = jnp.dot(w, jnp.ones((_C, 1), jnp.float32),
                preferred_element_type=jnp.float32)
    out_ref[...] = acc / s


def kernel(indices, query_points, codes_position, codes):
    return pl.pallas_call(
        _body,
        grid_spec=pltpu.PrefetchScalarGridSpec(
            num_scalar_prefetch=1,
            grid=(_N // _NB,),
            in_specs=[
                pl.BlockSpec((_NB, 3), lambda i, idx: (i, 0)),
                pl.BlockSpec((1, _C, 3), lambda i, idx: (idx[0], 0, 0)),
                pl.BlockSpec((1, _C, _D), lambda i, idx: (idx[0], 0, 0)),
            ],
            out_specs=pl.BlockSpec((_NB, _D), lambda i, idx: (i, 0)),
        ),
        out_shape=jax.ShapeDtypeStruct((_N, _D), jnp.float32),
    )(indices, query_points[0], codes_position, codes)
